# baseline (speedup 1.0000x reference)
import jax
import jax.numpy as jnp
from jax import lax
from jax.experimental import pallas as pl
from jax.experimental.pallas import tpu as pltpu
from jax.experimental.pallas import tpu_sc as plsc

_N = 10000
_D = 128
_DE = 16
_E = 320000

_CH = 32
_CPT = 160
_EPT = _CH * _CPT
_EHALF = _EPT * 32
_EPAD = 2 * _EHALF
_IDXR = _EPT // 128
_NPAD = 10240
_RPS = _NPAD // 16
_BE = 4096


_BR = _BE // 8
_ER8 = _E // 8


def _make_edge_proj_body(half):
    row0 = half * (_EHALF // 8)

    def _edge_proj_body(ea_ref, wb_ref, bb_ref, o_ref):
        i = pl.program_id(0)
        h = lax.dot_general(ea_ref[...], wb_ref[...], (((1,), (0,)), ((), ())),
                            preferred_element_type=jnp.float32) + bb_ref[...]
        rows = row0 + i * _BR + lax.broadcasted_iota(jnp.int32, (_BR, 1), 0)
        o_ref[...] = jnp.where(rows < _ER8, h, -1e30)

    return _edge_proj_body


def _edge_proj(ea2, W_big, b_big, half):
    last = (_ER8 - 1) // _BR
    blk0 = half * (_EHALF // _BE)
    return pl.pallas_call(
        _make_edge_proj_body(half),
        grid=(_EHALF // _BE,),
        in_specs=[
            pl.BlockSpec((_BR, _D),
                         lambda i: (jnp.minimum(blk0 + i, last), 0)),
            pl.BlockSpec((_D, 8 * _D), lambda i: (0, 0)),
            pl.BlockSpec((1, 8 * _D), lambda i: (0, 0)),
        ],
        out_specs=pl.BlockSpec((_BR, 8 * _D), lambda i: (i, 0)),
        out_shape=jax.ShapeDtypeStruct((_EHALF // 8, 8 * _D), jnp.float32),
    )(ea2, W_big, b_big)


def _sc_body(x_hbm, src_hbm, dst_hbm, e_hbm, z_hbm, out_hbm,
             src_all, dst_all, dstw0, dstw1, eidx0, eidx1,
             x_v0, x_v1, e_v0, e_v1, m_v0, m_v1,
             agg_sh, sem_x0, sem_x1, sem_e0, sem_e1, sem_s0, sem_s1):
    c = lax.axis_index("c")
    s = lax.axis_index("s")
    x_v = (x_v0, x_v1)
    e_v = (e_v0, e_v1)
    m_v = (m_v0, m_v1)
    dstw = (dstw0, dstw1)
    eidx = (eidx0, eidx1)
    sem_x = (sem_x0, sem_x1)
    sem_e = (sem_e0, sem_e1)
    sem_s = (sem_s0, sem_s1)

    wid = s * 2 + c
    tile8 = wid * (_EPT // 8)

    pltpu.sync_copy(src_hbm.at[pl.ds(wid * _IDXR, _IDXR)], src_all)
    pltpu.sync_copy(dst_hbm.at[pl.ds(wid * _IDXR, _IDXR)], dst_all)
    pltpu.sync_copy(z_hbm, agg_sh.at[pl.ds(s * _RPS, _RPS)])
    plsc.subcore_barrier()

    def src_slice(g):
        return src_all.at[g // 4, pl.ds((g % 4) * _CH, _CH)]

    def issue_loads(g, b):
        eidx[b][...] = (tile8 + g * (_CH // 8)) + lax.iota(jnp.int32, 16)
        pltpu.async_copy(e_hbm.at[eidx[b].at[pl.ds(0, _CH // 8)]],
                         e_v[b], sem_e[b])
        pltpu.async_copy(x_hbm.at[src_slice(g)], x_v[b], sem_x[b])

    def wait_loads(g, b):
        pltpu.make_async_copy(e_hbm.at[eidx[b].at[pl.ds(0, _CH // 8)]],
                              e_v[b], sem_e[b]).wait()
        pltpu.make_async_copy(x_hbm.at[src_slice(g)], x_v[b],
                              sem_x[b]).wait()

    def issue_scatter(b):
        pltpu.async_copy(m_v[b], agg_sh.at[dstw[b]], sem_s[b], add=True)

    def wait_scatter(b):
        pltpu.make_async_copy(m_v[b], agg_sh.at[dstw[b]],
                              sem_s[b]).wait()

    def copy_dst(g, b):
        for k in range(_CH // 16):
            dstw[b][pl.ds(k * 16, 16)] = (
                dst_all[g // 4, pl.ds((g % 4) * _CH + k * 16, 16)])

    def compute(b):
        xv, ev, mv = x_v[b], e_v[b], m_v[b]

        def row(r, carry):
            for db in range(_D // 16):
                sl = pl.ds(db * 16, 16)
                esl = pl.ds((r % 8) * _D + db * 16, 16)
                mv[r, sl] = jnp.maximum(xv[r, sl] + ev[r // 8, esl], 0.0)
            return carry

        lax.fori_loop(0, _CH, row, 0)

    def pair(g2, first, last):
        for b in (0, 1):
            g = 2 * g2 + b
            wait_loads(g, b)
            if not first:
                wait_scatter(b)
            copy_dst(g, b)
            compute(b)
            issue_scatter(b)
            if not last:
                issue_loads(g + 2, b)

    issue_loads(0, 0)
    issue_loads(1, 1)
    pair(0, True, False)

    def body(g2, carry):
        pair(g2, False, False)
        return carry

    lax.fori_loop(1, _CPT // 2 - 1, body, 0)
    pair(_CPT // 2 - 1, False, True)
    wait_scatter(0)
    wait_scatter(1)

    plsc.subcore_barrier()
    pltpu.sync_copy(agg_sh.at[pl.ds(s * _RPS, _RPS)],
                    out_hbm.at[c, pl.ds(s * _RPS, _RPS)])


_sc_agg = pl.kernel(
    _sc_body,
    mesh=plsc.VectorSubcoreMesh(core_axis_name="c", subcore_axis_name="s"),
    out_type=jax.ShapeDtypeStruct((2, _NPAD, _D), jnp.float32),
    scratch_types=[
        pltpu.VMEM((_IDXR, 128), jnp.int32),
        pltpu.VMEM((_IDXR, 128), jnp.int32),
        pltpu.VMEM((_CH,), jnp.int32),
        pltpu.VMEM((_CH,), jnp.int32),
        pltpu.VMEM((16,), jnp.int32),
        pltpu.VMEM((16,), jnp.int32),
        pltpu.VMEM((_CH, _D), jnp.float32),
        pltpu.VMEM((_CH, _D), jnp.float32),
        pltpu.VMEM((_CH // 8, 8 * _D), jnp.float32),
        pltpu.VMEM((_CH // 8, 8 * _D), jnp.float32),
        pltpu.VMEM((_CH, _D), jnp.float32),
        pltpu.VMEM((_CH, _D), jnp.float32),
        pltpu.VMEM_SHARED((_NPAD, _D), jnp.float32),
        pltpu.SemaphoreType.DMA,
        pltpu.SemaphoreType.DMA,
        pltpu.SemaphoreType.DMA,
        pltpu.SemaphoreType.DMA,
        pltpu.SemaphoreType.DMA,
        pltpu.SemaphoreType.DMA,
    ],
)


def _mlp_bn_body(x_ref, pa_ref, pb_ref, w1_ref, b1_ref, w2_ref, b2_ref,
                 g_ref, bt_ref, o_ref):
    agg = ((pa_ref[0, :_N, :] + pa_ref[1, :_N, :])
           + (pb_ref[0, :_N, :] + pb_ref[1, :_N, :]))
    h = x_ref[...] + agg
    h = lax.dot_general(h, w1_ref[...], (((1,), (1,)), ((), ())),
                        preferred_element_type=jnp.float32) + b1_ref[...]
    h = jnp.maximum(h, 0.0)
    h = lax.dot_general(h, w2_ref[...], (((1,), (1,)), ((), ())),
                        preferred_element_type=jnp.float32) + b2_ref[...]
    h = jnp.maximum(h, 0.0)
    mean = jnp.mean(h, axis=0, keepdims=True)
    var = jnp.mean(jnp.square(h - mean), axis=0, keepdims=True)
    o_ref[...] = (h - mean) * lax.rsqrt(var + 1e-5) * g_ref[...] + bt_ref[...]


def _mlp_bn(x, pa, pb, W1, b1, W2, b2, gamma, beta):
    return pl.pallas_call(
        _mlp_bn_body,
        out_shape=jax.ShapeDtypeStruct((_N, _D), jnp.float32),
    )(x, pa, pb, W1, b1.reshape(1, _D), W2, b2.reshape(1, _D),
      gamma.reshape(1, _D), beta.reshape(1, _D))


def kernel(x, edge_index, edge_attr, W_e, b_e, W1, b1, W2, b2, gamma, beta):
    src = edge_index[0]
    dst = edge_index[1]
    npad = _EPAD - _E
    fill = jnp.arange(npad, dtype=jnp.int32)
    src_f = jnp.concatenate([src, fill % _N])
    dst_f = jnp.concatenate([dst, fill % _N])
    zeros = jnp.zeros((_RPS, _D), jnp.float32)

    ea2 = edge_attr.reshape(_E // 8, _D)
    W_big = jnp.zeros((_D, 8 * _D), jnp.float32)
    for j in range(8):
        W_big = W_big.at[j * _DE:(j + 1) * _DE, j * _D:(j + 1) * _D].set(W_e.T)
    b_big = jnp.tile(b_e, 8).reshape(1, 8 * _D)
    e_a = _edge_proj(ea2, W_big, b_big, 0)
    e_b = _edge_proj(ea2, W_big, b_big, 1)
    parts = []
    for k, e_k in ((0, e_a), (1, e_b)):
        src_k = lax.dynamic_slice_in_dim(src_f, k * _EHALF, _EHALF)
        dst_k = lax.dynamic_slice_in_dim(dst_f, k * _EHALF, _EHALF)
        parts.append(_sc_agg(x, src_k.reshape(32 * _IDXR, 128),
                             dst_k.reshape(32 * _IDXR, 128), e_k, zeros))
    return _mlp_bn(x, parts[0], parts[1], W1, b1, W2, b2, gamma, beta)


# speedup vs baseline: 1.5457x; 1.5457x over previous
import jax
import jax.numpy as jnp
from jax import lax
from jax.experimental import pallas as pl
from jax.experimental.pallas import tpu as pltpu
from jax.experimental.pallas import tpu_sc as plsc

_N = 10000
_D = 128
_DE = 16
_E = 320000

_CH = 32
_CPT = 160
_EPT = _CH * _CPT
_EHALF = _EPT * 32
_EPAD = 2 * _EHALF
_IDXR = _EPT // 128
_NPAD = 10240
_RPS = _NPAD // 16
_BE = 4096


def _make_edge_proj_body(half):
    row0 = half * _EHALF

    def _edge_proj_body(ea_ref, we_ref, be_ref, o_ref):
        i = pl.program_id(0)
        h = lax.dot_general(ea_ref[...], we_ref[...], (((1,), (1,)), ((), ())),
                            preferred_element_type=jnp.float32) + be_ref[...]
        rows = row0 + i * _BE + lax.broadcasted_iota(jnp.int32, (_BE, 1), 0)
        o_ref[...] = jnp.where(rows < _E, h, -1e30)

    return _edge_proj_body


def _edge_proj(ea, W_e, b_e, half):
    last = (_E - 1) // _BE
    blk0 = half * (_EHALF // _BE)
    return pl.pallas_call(
        _make_edge_proj_body(half),
        grid=(_EHALF // _BE,),
        in_specs=[
            pl.BlockSpec((_BE, _DE),
                         lambda i: (jnp.minimum(blk0 + i, last), 0)),
            pl.BlockSpec((_D, _DE), lambda i: (0, 0)),
            pl.BlockSpec((1, _D), lambda i: (0, 0)),
        ],
        out_specs=pl.BlockSpec((_BE, _D), lambda i: (i, 0)),
        out_shape=jax.ShapeDtypeStruct((_EHALF, _D), jnp.float32),
    )(ea, W_e, b_e.reshape(1, _D))


def _sc_body(x_hbm, src_hbm, dst_hbm, e_hbm, z_hbm, out_hbm,
             src_all, dst_all, dstw0, dstw1, x_v0, x_v1, e_v0, e_v1, m_v0, m_v1,
             agg_sh, sem_x0, sem_x1, sem_e0, sem_e1, sem_s0, sem_s1):
    c = lax.axis_index("c")
    s = lax.axis_index("s")
    x_v = (x_v0, x_v1)
    e_v = (e_v0, e_v1)
    m_v = (m_v0, m_v1)
    dstw = (dstw0, dstw1)
    sem_x = (sem_x0, sem_x1)
    sem_e = (sem_e0, sem_e1)
    sem_s = (sem_s0, sem_s1)

    wid = s * 2 + c
    tile_base = wid * _EPT

    pltpu.sync_copy(src_hbm.at[pl.ds(wid * _IDXR, _IDXR)], src_all)
    pltpu.sync_copy(dst_hbm.at[pl.ds(wid * _IDXR, _IDXR)], dst_all)
    pltpu.sync_copy(z_hbm, agg_sh.at[pl.ds(s * _RPS, _RPS)])
    plsc.subcore_barrier()

    def src_slice(g):
        return src_all.at[g // 4, pl.ds((g % 4) * _CH, _CH)]

    def issue_loads(g, b):
        pltpu.async_copy(e_hbm.at[pl.ds(tile_base + g * _CH, _CH)],
                         e_v[b], sem_e[b])
        pltpu.async_copy(x_hbm.at[src_slice(g)], x_v[b], sem_x[b])

    def wait_loads(g, b):
        pltpu.make_async_copy(e_hbm.at[pl.ds(tile_base + g * _CH, _CH)],
                              e_v[b], sem_e[b]).wait()
        pltpu.make_async_copy(x_hbm.at[src_slice(g)], x_v[b],
                              sem_x[b]).wait()

    def issue_scatter(b):
        pltpu.async_copy(m_v[b], agg_sh.at[dstw[b]], sem_s[b], add=True)

    def wait_scatter(b):
        pltpu.make_async_copy(m_v[b], agg_sh.at[dstw[b]],
                              sem_s[b]).wait()

    def copy_dst(g, b):
        for k in range(_CH // 16):
            dstw[b][pl.ds(k * 16, 16)] = (
                dst_all[g // 4, pl.ds((g % 4) * _CH + k * 16, 16)])

    def compute(b):
        xv, ev, mv = x_v[b], e_v[b], m_v[b]

        def row(r, carry):
            for db in range(_D // 16):
                sl = pl.ds(db * 16, 16)
                mv[r, sl] = jnp.maximum(xv[r, sl] + ev[r, sl], 0.0)
            return carry

        lax.fori_loop(0, _CH, row, 0)

    def pair(g2, first, last):
        for b in (0, 1):
            g = 2 * g2 + b
            wait_loads(g, b)
            if not first:
                wait_scatter(b)
            copy_dst(g, b)
            compute(b)
            issue_scatter(b)
            if not last:
                issue_loads(g + 2, b)

    issue_loads(0, 0)
    issue_loads(1, 1)
    pair(0, True, False)

    def body(g2, carry):
        pair(g2, False, False)
        return carry

    lax.fori_loop(1, _CPT // 2 - 1, body, 0)
    pair(_CPT // 2 - 1, False, True)
    wait_scatter(0)
    wait_scatter(1)

    plsc.subcore_barrier()
    pltpu.sync_copy(agg_sh.at[pl.ds(s * _RPS, _RPS)],
                    out_hbm.at[c, pl.ds(s * _RPS, _RPS)])


_sc_agg = pl.kernel(
    _sc_body,
    mesh=plsc.VectorSubcoreMesh(core_axis_name="c", subcore_axis_name="s"),
    out_type=jax.ShapeDtypeStruct((2, _NPAD, _D), jnp.float32),
    scratch_types=[
        pltpu.VMEM((_IDXR, 128), jnp.int32),
        pltpu.VMEM((_IDXR, 128), jnp.int32),
        pltpu.VMEM((_CH,), jnp.int32),
        pltpu.VMEM((_CH,), jnp.int32),
        pltpu.VMEM((_CH, _D), jnp.float32),
        pltpu.VMEM((_CH, _D), jnp.float32),
        pltpu.VMEM((_CH, _D), jnp.float32),
        pltpu.VMEM((_CH, _D), jnp.float32),
        pltpu.VMEM((_CH, _D), jnp.float32),
        pltpu.VMEM((_CH, _D), jnp.float32),
        pltpu.VMEM_SHARED((_NPAD, _D), jnp.float32),
        pltpu.SemaphoreType.DMA,
        pltpu.SemaphoreType.DMA,
        pltpu.SemaphoreType.DMA,
        pltpu.SemaphoreType.DMA,
        pltpu.SemaphoreType.DMA,
        pltpu.SemaphoreType.DMA,
    ],
)


def _mlp_bn_body(x_ref, pa_ref, pb_ref, w1_ref, b1_ref, w2_ref, b2_ref,
                 g_ref, bt_ref, o_ref):
    agg = ((pa_ref[0, :_N, :] + pa_ref[1, :_N, :])
           + (pb_ref[0, :_N, :] + pb_ref[1, :_N, :]))
    h = x_ref[...] + agg
    h = lax.dot_general(h, w1_ref[...], (((1,), (1,)), ((), ())),
                        preferred_element_type=jnp.float32) + b1_ref[...]
    h = jnp.maximum(h, 0.0)
    h = lax.dot_general(h, w2_ref[...], (((1,), (1,)), ((), ())),
                        preferred_element_type=jnp.float32) + b2_ref[...]
    h = jnp.maximum(h, 0.0)
    mean = jnp.mean(h, axis=0, keepdims=True)
    var = jnp.mean(jnp.square(h - mean), axis=0, keepdims=True)
    o_ref[...] = (h - mean) * lax.rsqrt(var + 1e-5) * g_ref[...] + bt_ref[...]


def _mlp_bn(x, pa, pb, W1, b1, W2, b2, gamma, beta):
    return pl.pallas_call(
        _mlp_bn_body,
        out_shape=jax.ShapeDtypeStruct((_N, _D), jnp.float32),
    )(x, pa, pb, W1, b1.reshape(1, _D), W2, b2.reshape(1, _D),
      gamma.reshape(1, _D), beta.reshape(1, _D))


def kernel(x, edge_index, edge_attr, W_e, b_e, W1, b1, W2, b2, gamma, beta):
    src = edge_index[0]
    dst = edge_index[1]
    npad = _EPAD - _E
    fill = jnp.arange(npad, dtype=jnp.int32)
    src_f = jnp.concatenate([src, fill % _N])
    dst_f = jnp.concatenate([dst, fill % _N])
    zeros = jnp.zeros((_RPS, _D), jnp.float32)

    e_a = _edge_proj(edge_attr, W_e, b_e, 0)
    e_b = _edge_proj(edge_attr, W_e, b_e, 1)
    parts = []
    for k, e_k in ((0, e_a), (1, e_b)):
        src_k = lax.dynamic_slice_in_dim(src_f, k * _EHALF, _EHALF)
        dst_k = lax.dynamic_slice_in_dim(dst_f, k * _EHALF, _EHALF)
        parts.append(_sc_agg(x, src_k.reshape(32 * _IDXR, 128),
                             dst_k.reshape(32 * _IDXR, 128), e_k, zeros))
    return _mlp_bn(x, parts[0], parts[1], W1, b1, W2, b2, gamma, beta)
